# baseline (speedup 1.0000x reference)
import jax
import jax.numpy as jnp
from jax import lax
from jax.experimental import pallas as pl
from jax.experimental.pallas import tpu as pltpu
from jax.experimental.pallas import tpu_sc as plsc
import functools

N = 100000
G = 20000
G_PAD = 20480
ROWS = 784
NP = ROWS * 128
NW = 16
RPW = ROWS // NW
ZPW = G_PAD // NW
CHUNK = RPW * 128
EPS = 1e-10


def _sc_body(now_hbm, it_hbm, mi_hbm, if_hbm, i0, i1, i2, i3, i4, i5,
             out0, out1, tr_out,
             ix0, ix1, ix2, tv, cv, gt, gu, bb, sv, zv, nt, nc, nb,
             tt0, tt1, tt2, tc0, tc1, tc2, sem):
    cid = lax.axis_index("c")
    sid = lax.axis_index("s")
    out_refs = [out0, out1]
    ids_refs = [i0, i1, i2, i3, i4, i5]
    ixs = [ix0, ix1, ix2]
    tts = [tt0, tt1, tt2]
    tcs = [tc0, tc1, tc2]
    zsl = pl.ds(sid * ZPW, ZPW)

    pltpu.sync_copy(now_hbm, nb)
    pltpu.sync_copy(it_hbm.at[sid], gt)
    pltpu.sync_copy(mi_hbm.at[sid], gu)
    pltpu.sync_copy(if_hbm.at[sid], bb)
    for c in range(2):
        @pl.when(cid == c)
        def _(c=c):
            for q in range(3):
                pltpu.sync_copy(ids_refs[3 * c + q].at[sid], ixs[q])

    nvec = nb[pl.ds(0, 16)]
    base = sid * CHUNK

    def _tb(j, c):
        for kk in range(8):
            o = kk * 16
            t = jnp.maximum(nvec - gt[j, pl.ds(o, 16)] * 10.0 - 1.0, 0.0)
            tv[j, pl.ds(o, 16)] = (bb[j, pl.ds(o, 16)] * gu[j, pl.ds(o, 16)]
                                   * (t * t) * jnp.exp(-0.5 * t))
            gi = base + j * 128 + o + lax.iota(jnp.int32, 16)
            cv[j, pl.ds(o, 16)] = jnp.where(gi < N, 1.0, 0.0)
            sv[j, pl.ds(o, 16)] = jnp.zeros((16,), jnp.float32)
        return c
    lax.fori_loop(0, RPW, _tb, 0)

    @pl.when(cid == 0)
    def _():
        pltpu.sync_copy(tv, tr_out.at[sid])

    def _zb(i, c):
        zv[pl.ds(i * 16, 16)] = jnp.zeros((16,), jnp.float32)
        return c
    lax.fori_loop(0, ZPW // 16, _zb, 0)

    for q in range(3):
        pltpu.sync_copy(zv, tts[q].at[zsl])
        pltpu.sync_copy(zv, tcs[q].at[zsl])
    plsc.subcore_barrier()

    for q in range(3):
        def _sf(j, c, q=q):
            pltpu.async_copy(tv.at[j], tts[q].at[ixs[q].at[j]], sem,
                             add=True)
            pltpu.async_copy(cv.at[j], tcs[q].at[ixs[q].at[j]], sem,
                             add=True)
            return c
        lax.fori_loop(0, RPW, _sf, 0)
    for q in range(3):
        def _sd(j, c, q=q):
            pltpu.make_async_copy(tv.at[j], tts[q].at[ixs[q].at[j]],
                                  sem).wait()
            pltpu.make_async_copy(cv.at[j], tcs[q].at[ixs[q].at[j]],
                                  sem).wait()
            return c
        lax.fori_loop(0, RPW, _sd, 0)
    plsc.subcore_barrier()

    for q in range(3):
        pltpu.sync_copy(tts[q].at[zsl], nt)
        pltpu.sync_copy(tcs[q].at[zsl], nc)

        def _nb2(i, c):
            o = i * 16
            nt[pl.ds(o, 16)] = (nt[pl.ds(o, 16)]
                                / jnp.maximum(nc[pl.ds(o, 16)], 1.0))
            return c
        lax.fori_loop(0, ZPW // 16, _nb2, 0)
        pltpu.sync_copy(nt, tts[q].at[zsl])
    plsc.subcore_barrier()

    gbufs = [gt, gu, gt]

    def _gfire(q):
        def _gf(j, c, q=q):
            pltpu.async_copy(tts[q].at[ixs[q].at[j]], gbufs[q].at[j], sem)
            return c
        lax.fori_loop(0, RPW, _gf, 0)

    def _gdrain(q):
        def _gd(j, c, q=q):
            pltpu.make_async_copy(tts[q].at[ixs[q].at[j]], gbufs[q].at[j],
                                  sem).wait()
            return c
        lax.fori_loop(0, RPW, _gd, 0)

    def _gaccum(q):
        def _ab(j, c, q=q):
            for kk in range(8):
                o = kk * 16
                sv[j, pl.ds(o, 16)] = (sv[j, pl.ds(o, 16)]
                                       + gbufs[q][j, pl.ds(o, 16)])
            return c
        lax.fori_loop(0, RPW, _ab, 0)

    _gfire(0)
    _gfire(1)
    _gdrain(0)
    _gaccum(0)
    _gfire(2)
    _gdrain(1)
    _gaccum(1)
    _gdrain(2)
    _gaccum(2)

    for c in range(2):
        @pl.when(cid == c)
        def _(c=c):
            pltpu.sync_copy(sv, out_refs[c].at[sid])


_sc_call = functools.partial(
    pl.kernel,
    out_type=[jax.ShapeDtypeStruct((NW, RPW, 128), jnp.float32),
              jax.ShapeDtypeStruct((NW, RPW, 128), jnp.float32),
              jax.ShapeDtypeStruct((NW, RPW, 128), jnp.float32)],
    mesh=plsc.VectorSubcoreMesh(core_axis_name="c", subcore_axis_name="s"),
    scratch_types=[
        pltpu.VMEM((RPW, 128), jnp.int32),
        pltpu.VMEM((RPW, 128), jnp.int32),
        pltpu.VMEM((RPW, 128), jnp.int32),
        pltpu.VMEM((RPW, 128), jnp.float32),
        pltpu.VMEM((RPW, 128), jnp.float32),
        pltpu.VMEM((RPW, 128), jnp.float32),
        pltpu.VMEM((RPW, 128), jnp.float32),
        pltpu.VMEM((RPW, 128), jnp.float32),
        pltpu.VMEM((RPW, 128), jnp.float32),
        pltpu.VMEM((ZPW,), jnp.float32),
        pltpu.VMEM((ZPW,), jnp.float32),
        pltpu.VMEM((ZPW,), jnp.float32),
        pltpu.VMEM((16,), jnp.float32),
        pltpu.VMEM_SHARED((G_PAD,), jnp.float32),
        pltpu.VMEM_SHARED((G_PAD,), jnp.float32),
        pltpu.VMEM_SHARED((G_PAD,), jnp.float32),
        pltpu.VMEM_SHARED((G_PAD,), jnp.float32),
        pltpu.VMEM_SHARED((G_PAD,), jnp.float32),
        pltpu.VMEM_SHARED((G_PAD,), jnp.float32),
        pltpu.SemaphoreType.DMA,
    ],
)(_sc_body)


def _finish_body(tr, s0, s1, su, ii, u0, u1, o):
    trans = tr[...]
    isf = ii[...]
    susc = su[...]
    logp = -(susc * (s0[...] + s1[...]))
    p = jnp.exp(logp)
    a0 = jnp.log(p + EPS)
    a1 = jnp.log(1.0 - p + EPS)
    g0 = -jnp.log(-jnp.log(u0[...] + EPS) + EPS)
    g1 = -jnp.log(-jnp.log(u1[...] + EPS) + EPS)
    arg = (a1 - a0 + g1 - g0) * 10.0
    soft1 = 1.0 / (1.0 + jnp.exp(-arg))
    new_inf = soft1 * (1.0 - isf)
    o[0] = trans
    o[1] = p
    o[2] = new_inf
    o[3] = jnp.maximum(0.0, susc - new_inf)
    new_isinf = isf + new_inf
    o[4] = new_isinf
    o[5] = new_isinf * (1.0 / (1.0 + jnp.exp(-(trans - 1.0))))


_finish_call = pl.pallas_call(
    _finish_body,
    out_shape=jax.ShapeDtypeStruct((6, ROWS, 128), jnp.float32),
)


def _pad2d(x):
    return jnp.pad(x, (0, NP - N)).reshape(ROWS, 128)


def _pad3d(x):
    return jnp.pad(x, (0, NP - N)).reshape(NW, RPW, 128)


def kernel(susceptibility, infection_time, max_infectiousness, is_infected,
           company_ids, school_ids, household_ids, university_ids,
           leisure_ids, care_home_ids, now):
    now16 = jnp.full((16,), jnp.asarray(now, jnp.float32))
    isf = is_infected.astype(jnp.float32)
    ids3 = [_pad3d(i).astype(jnp.int32)
            for i in (company_ids, school_ids, household_ids,
                      university_ids, leisure_ids, care_home_ids)]
    u = jax.random.uniform(jax.random.key(42), (N, 2), dtype=jnp.float32)

    s0, s1, tr3 = _sc_call(now16, _pad3d(infection_time),
                           _pad3d(max_infectiousness), _pad3d(isf), *ids3)
    out = _finish_call(tr3.reshape(ROWS, 128), s0.reshape(ROWS, 128),
                       s1.reshape(ROWS, 128), _pad2d(susceptibility),
                       _pad2d(isf), _pad2d(u[:, 0]), _pad2d(u[:, 1]))
    return out.reshape(6, NP)[:, :N]


# speedup vs baseline: 1.2734x; 1.2734x over previous
import jax
import jax.numpy as jnp
from jax import lax
from jax.experimental import pallas as pl
from jax.experimental.pallas import tpu as pltpu
from jax.experimental.pallas import tpu_sc as plsc
import functools

N = 100000
G = 20000
G_PAD = 20480
ROWS = 784
NP = ROWS * 128
NW = 16
RPW = ROWS // NW
ZPW = G_PAD // NW
EPS = 1e-10


def _trans_body(now_ref, it_ref, mi_ref, ii_ref, o_ref):
    t = jnp.maximum(now_ref[0, 0] - it_ref[...] * 10.0 - 1.0, 0.0)
    o_ref[...] = ii_ref[...] * mi_ref[...] * (t * t) * jnp.exp(-0.5 * t)


_trans_call = pl.pallas_call(
    _trans_body,
    out_shape=jax.ShapeDtypeStruct((NW, RPW, 128), jnp.float32),
    in_specs=[
        pl.BlockSpec(memory_space=pltpu.SMEM),
        pl.BlockSpec(),
        pl.BlockSpec(),
        pl.BlockSpec(),
    ],
)


CHUNK = RPW * 128


BURST = 7


def _sc_body(tr_hbm, cw_hbm, ids_hbm, out0, out1,
             idx2, tv, cv, gt, sv, zv, nt, nc, tt_sh, tc_sh, sem):
    cid = lax.axis_index("c")
    sid = lax.axis_index("s")
    out_refs = [out0, out1]

    pltpu.sync_copy(tr_hbm.at[sid], tv)
    pltpu.sync_copy(cw_hbm.at[sid], cv)

    def _zb(i, c):
        zv[pl.ds(i * 16, 16)] = jnp.zeros((16,), jnp.float32)
        return c
    lax.fori_loop(0, ZPW // 16, _zb, 0)

    def _za(j, c):
        for kk in range(8):
            sv[j, pl.ds(kk * 16, 16)] = jnp.zeros((16,), jnp.float32)
        return c
    lax.fori_loop(0, RPW, _za, 0)

    def _round(r, carry):
        pltpu.sync_copy(zv, tt_sh.at[pl.ds(sid * ZPW, ZPW)])
        pltpu.sync_copy(zv, tc_sh.at[pl.ds(sid * ZPW, ZPW)])
        plsc.subcore_barrier()

        v = 3 * cid + r
        pltpu.sync_copy(ids_hbm.at[v, sid], idx2)

        def _sf(j, c):
            pltpu.async_copy(tv.at[j], tt_sh.at[idx2.at[j]], sem, add=True)
            pltpu.async_copy(cv.at[j], tc_sh.at[idx2.at[j]], sem, add=True)
            return c
        lax.fori_loop(0, RPW, _sf, 0)

        def _sd(j, c):
            pltpu.make_async_copy(tv.at[j], tt_sh.at[idx2.at[j]], sem).wait()
            pltpu.make_async_copy(cv.at[j], tc_sh.at[idx2.at[j]], sem).wait()
            return c
        lax.fori_loop(0, RPW, _sd, 0)
        plsc.subcore_barrier()

        pltpu.sync_copy(tt_sh.at[pl.ds(sid * ZPW, ZPW)], nt)
        pltpu.sync_copy(tc_sh.at[pl.ds(sid * ZPW, ZPW)], nc)

        def _nb(i, c):
            o = i * 16
            nt[pl.ds(o, 16)] = (nt[pl.ds(o, 16)]
                                / jnp.maximum(nc[pl.ds(o, 16)], 1.0))
            return c
        lax.fori_loop(0, ZPW // 16, _nb, 0)
        pltpu.sync_copy(nt, tt_sh.at[pl.ds(sid * ZPW, ZPW)])
        plsc.subcore_barrier()

        def _gf(j, c):
            pltpu.async_copy(tt_sh.at[idx2.at[j]], gt.at[j], sem)
            return c
        lax.fori_loop(0, RPW, _gf, 0)

        def _gd(j, c):
            pltpu.make_async_copy(tt_sh.at[idx2.at[j]], gt.at[j], sem).wait()
            return c
        lax.fori_loop(0, RPW, _gd, 0)

        def _ab(j, c):
            for kk in range(8):
                o = kk * 16
                sv[j, pl.ds(o, 16)] = (sv[j, pl.ds(o, 16)]
                                       + gt[j, pl.ds(o, 16)])
            return c
        lax.fori_loop(0, RPW, _ab, 0)
        plsc.subcore_barrier()
        return carry

    lax.fori_loop(0, 3, _round, 0)

    for c in range(2):
        @pl.when(cid == c)
        def _(c=c):
            pltpu.sync_copy(sv, out_refs[c].at[sid])


_sc_call = functools.partial(
    pl.kernel,
    out_type=[jax.ShapeDtypeStruct((NW, RPW, 128), jnp.float32),
              jax.ShapeDtypeStruct((NW, RPW, 128), jnp.float32)],
    mesh=plsc.VectorSubcoreMesh(core_axis_name="c", subcore_axis_name="s"),
    scratch_types=[
        pltpu.VMEM((RPW, 128), jnp.int32),
        pltpu.VMEM((RPW, 128), jnp.float32),
        pltpu.VMEM((RPW, 128), jnp.float32),
        pltpu.VMEM((RPW, 128), jnp.float32),
        pltpu.VMEM((RPW, 128), jnp.float32),
        pltpu.VMEM((ZPW,), jnp.float32),
        pltpu.VMEM((ZPW,), jnp.float32),
        pltpu.VMEM((ZPW,), jnp.float32),
        pltpu.VMEM_SHARED((G_PAD,), jnp.float32),
        pltpu.VMEM_SHARED((G_PAD,), jnp.float32),
        pltpu.SemaphoreType.DMA,
    ],
)(_sc_body)


def _finish_body(tr, s0, s1, su, ii, u0, u1, o):
    trans = tr[...]
    isf = ii[...]
    susc = su[...]
    logp = -(susc * (s0[...] + s1[...]))
    p = jnp.exp(logp)
    a0 = jnp.log(p + EPS)
    a1 = jnp.log(1.0 - p + EPS)
    g0 = -jnp.log(-jnp.log(u0[...] + EPS) + EPS)
    g1 = -jnp.log(-jnp.log(u1[...] + EPS) + EPS)
    arg = (a1 - a0 + g1 - g0) * 10.0
    soft1 = 1.0 / (1.0 + jnp.exp(-arg))
    new_inf = soft1 * (1.0 - isf)
    o[0] = trans
    o[1] = p
    o[2] = new_inf
    o[3] = jnp.maximum(0.0, susc - new_inf)
    new_isinf = isf + new_inf
    o[4] = new_isinf
    o[5] = new_isinf * (1.0 / (1.0 + jnp.exp(-(trans - 1.0))))


_finish_call = pl.pallas_call(
    _finish_body,
    out_shape=jax.ShapeDtypeStruct((6, NW, RPW, 128), jnp.float32),
)


def _pad3d(x):
    return jnp.pad(x, (0, NP - N)).reshape(NW, RPW, 128)


def kernel(susceptibility, infection_time, max_infectiousness, is_infected,
           company_ids, school_ids, household_ids, university_ids,
           leisure_ids, care_home_ids, now):
    now_f = jnp.asarray(now, jnp.float32).reshape(1, 1)
    isf = _pad3d(is_infected.astype(jnp.float32))
    it3 = _pad3d(infection_time)
    mi3 = _pad3d(max_infectiousness)
    su3 = _pad3d(susceptibility)
    ids4 = jnp.stack(
        [jnp.pad(i, (0, NP - N)) for i in
         (company_ids, school_ids, household_ids,
          university_ids, leisure_ids, care_home_ids)]
    ).astype(jnp.int32).reshape(6, NW, RPW, 128)
    cw3 = jnp.where(jnp.arange(NP) < N, 1.0, 0.0).astype(
        jnp.float32).reshape(NW, RPW, 128)
    u = jax.random.uniform(jax.random.key(42), (N, 2), dtype=jnp.float32)
    u0 = _pad3d(u[:, 0])
    u1 = _pad3d(u[:, 1])

    trans3 = _trans_call(now_f, it3, mi3, isf)
    s0, s1 = _sc_call(trans3, cw3, ids4)
    out = _finish_call(trans3, s0, s1, su3, isf, u0, u1)
    return out.reshape(6, NP)[:, :N]


# speedup vs baseline: 1.2914x; 1.0141x over previous
import jax
import jax.numpy as jnp
from jax import lax
from jax.experimental import pallas as pl
from jax.experimental.pallas import tpu as pltpu
from jax.experimental.pallas import tpu_sc as plsc
import functools

N = 100000
G = 20000
G_PAD = 20480
ROWS = 784
NP = ROWS * 128
NW = 16
RPW = ROWS // NW
ZPW = G_PAD // NW
EPS = 1e-10


def _trans_body(now_ref, it_ref, mi_ref, ii_ref, o_ref):
    t = jnp.maximum(now_ref[0, 0] - it_ref[...] * 10.0 - 1.0, 0.0)
    o_ref[...] = ii_ref[...] * mi_ref[...] * (t * t) * jnp.exp(-0.5 * t)


_trans_call = pl.pallas_call(
    _trans_body,
    out_shape=jax.ShapeDtypeStruct((NW, RPW, 128), jnp.float32),
    in_specs=[
        pl.BlockSpec(memory_space=pltpu.SMEM),
        pl.BlockSpec(),
        pl.BlockSpec(),
        pl.BlockSpec(),
    ],
)


CHUNK = RPW * 128


BURST = 7


def _sc_body(tr_hbm, cw_hbm, ids_hbm, out0, out1,
             idx2, tv, cv, gt, sv, zv, nt, nc, tt_sh, tc_sh, sem):
    cid = lax.axis_index("c")
    sid = lax.axis_index("s")
    out_refs = [out0, out1]

    sa = pltpu.async_copy(tr_hbm.at[sid], tv, sem)
    sb = pltpu.async_copy(cw_hbm.at[sid], cv, sem)
    sa.wait()
    sb.wait()

    def _zb(i, c):
        zv[pl.ds(i * 16, 16)] = jnp.zeros((16,), jnp.float32)
        return c
    lax.fori_loop(0, ZPW // 16, _zb, 0)

    def _za(j, c):
        for kk in range(8):
            sv[j, pl.ds(kk * 16, 16)] = jnp.zeros((16,), jnp.float32)
        return c
    lax.fori_loop(0, RPW, _za, 0)

    def _round(r, carry):
        za = pltpu.async_copy(zv, tt_sh.at[pl.ds(sid * ZPW, ZPW)], sem)
        zb = pltpu.async_copy(zv, tc_sh.at[pl.ds(sid * ZPW, ZPW)], sem)
        za.wait()
        zb.wait()
        plsc.subcore_barrier()

        v = 3 * cid + r
        pltpu.sync_copy(ids_hbm.at[v, sid], idx2)

        def _sf(j, c):
            pltpu.async_copy(tv.at[j], tt_sh.at[idx2.at[j]], sem, add=True)
            pltpu.async_copy(cv.at[j], tc_sh.at[idx2.at[j]], sem, add=True)
            return c
        lax.fori_loop(0, RPW, _sf, 0)

        def _sd(j, c):
            pltpu.make_async_copy(tv.at[j], tt_sh.at[idx2.at[j]], sem).wait()
            pltpu.make_async_copy(cv.at[j], tc_sh.at[idx2.at[j]], sem).wait()
            return c
        lax.fori_loop(0, RPW, _sd, 0)
        plsc.subcore_barrier()

        na = pltpu.async_copy(tt_sh.at[pl.ds(sid * ZPW, ZPW)], nt, sem)
        nb = pltpu.async_copy(tc_sh.at[pl.ds(sid * ZPW, ZPW)], nc, sem)
        na.wait()
        nb.wait()

        def _nb(i, c):
            o = i * 16
            nt[pl.ds(o, 16)] = (nt[pl.ds(o, 16)]
                                / jnp.maximum(nc[pl.ds(o, 16)], 1.0))
            return c
        lax.fori_loop(0, ZPW // 16, _nb, 0)
        pltpu.sync_copy(nt, tt_sh.at[pl.ds(sid * ZPW, ZPW)])
        plsc.subcore_barrier()

        def _gf(j, c):
            pltpu.async_copy(tt_sh.at[idx2.at[j]], gt.at[j], sem)
            return c
        lax.fori_loop(0, RPW, _gf, 0)

        def _gd(j, c):
            pltpu.make_async_copy(tt_sh.at[idx2.at[j]], gt.at[j], sem).wait()
            return c
        lax.fori_loop(0, RPW, _gd, 0)

        def _ab(j, c):
            for kk in range(8):
                o = kk * 16
                sv[j, pl.ds(o, 16)] = (sv[j, pl.ds(o, 16)]
                                       + gt[j, pl.ds(o, 16)])
            return c
        lax.fori_loop(0, RPW, _ab, 0)
        plsc.subcore_barrier()
        return carry

    lax.fori_loop(0, 3, _round, 0)

    for c in range(2):
        @pl.when(cid == c)
        def _(c=c):
            pltpu.sync_copy(sv, out_refs[c].at[sid])


_sc_call = functools.partial(
    pl.kernel,
    out_type=[jax.ShapeDtypeStruct((NW, RPW, 128), jnp.float32),
              jax.ShapeDtypeStruct((NW, RPW, 128), jnp.float32)],
    mesh=plsc.VectorSubcoreMesh(core_axis_name="c", subcore_axis_name="s"),
    scratch_types=[
        pltpu.VMEM((RPW, 128), jnp.int32),
        pltpu.VMEM((RPW, 128), jnp.float32),
        pltpu.VMEM((RPW, 128), jnp.float32),
        pltpu.VMEM((RPW, 128), jnp.float32),
        pltpu.VMEM((RPW, 128), jnp.float32),
        pltpu.VMEM((ZPW,), jnp.float32),
        pltpu.VMEM((ZPW,), jnp.float32),
        pltpu.VMEM((ZPW,), jnp.float32),
        pltpu.VMEM_SHARED((G_PAD,), jnp.float32),
        pltpu.VMEM_SHARED((G_PAD,), jnp.float32),
        pltpu.SemaphoreType.DMA,
    ],
)(_sc_body)


def _finish_body(tr, s0, s1, su, ii, u0, u1, o):
    trans = tr[...]
    isf = ii[...]
    susc = su[...]
    logp = -(susc * (s0[...] + s1[...]))
    p = jnp.exp(logp)
    a0 = jnp.log(p + EPS)
    a1 = jnp.log(1.0 - p + EPS)
    g0 = -jnp.log(-jnp.log(u0[...] + EPS) + EPS)
    g1 = -jnp.log(-jnp.log(u1[...] + EPS) + EPS)
    arg = (a1 - a0 + g1 - g0) * 10.0
    soft1 = 1.0 / (1.0 + jnp.exp(-arg))
    new_inf = soft1 * (1.0 - isf)
    o[0] = trans
    o[1] = p
    o[2] = new_inf
    o[3] = jnp.maximum(0.0, susc - new_inf)
    new_isinf = isf + new_inf
    o[4] = new_isinf
    o[5] = new_isinf * (1.0 / (1.0 + jnp.exp(-(trans - 1.0))))


_finish_call = pl.pallas_call(
    _finish_body,
    out_shape=jax.ShapeDtypeStruct((6, NW, RPW, 128), jnp.float32),
)


def _pad3d(x):
    return jnp.pad(x, (0, NP - N)).reshape(NW, RPW, 128)


def kernel(susceptibility, infection_time, max_infectiousness, is_infected,
           company_ids, school_ids, household_ids, university_ids,
           leisure_ids, care_home_ids, now):
    now_f = jnp.asarray(now, jnp.float32).reshape(1, 1)
    isf = _pad3d(is_infected.astype(jnp.float32))
    it3 = _pad3d(infection_time)
    mi3 = _pad3d(max_infectiousness)
    su3 = _pad3d(susceptibility)
    ids4 = jnp.stack(
        [jnp.pad(i, (0, NP - N)) for i in
         (company_ids, school_ids, household_ids,
          university_ids, leisure_ids, care_home_ids)]
    ).astype(jnp.int32).reshape(6, NW, RPW, 128)
    cw3 = jnp.where(jnp.arange(NP) < N, 1.0, 0.0).astype(
        jnp.float32).reshape(NW, RPW, 128)
    u = jax.random.uniform(jax.random.key(42), (N, 2), dtype=jnp.float32)
    u0 = _pad3d(u[:, 0])
    u1 = _pad3d(u[:, 1])

    trans3 = _trans_call(now_f, it3, mi3, isf)
    s0, s1 = _sc_call(trans3, cw3, ids4)
    out = _finish_call(trans3, s0, s1, su3, isf, u0, u1)
    return out.reshape(6, NP)[:, :N]


# speedup vs baseline: 1.2920x; 1.0005x over previous
import jax
import jax.numpy as jnp
from jax import lax
from jax.experimental import pallas as pl
from jax.experimental.pallas import tpu as pltpu
from jax.experimental.pallas import tpu_sc as plsc
import functools

N = 100000
G = 20000
G_PAD = 20480
ROWS = 784
NP = ROWS * 128
NW = 16
RPW = ROWS // NW
ZPW = G_PAD // NW
EPS = 1e-10


def _trans_body(now_ref, it_ref, mi_ref, ii_ref, o_ref):
    t = jnp.maximum(now_ref[0, 0] - it_ref[...] * 10.0 - 1.0, 0.0)
    o_ref[...] = ii_ref[...] * mi_ref[...] * (t * t) * jnp.exp(-0.5 * t)


_trans_call = pl.pallas_call(
    _trans_body,
    out_shape=jax.ShapeDtypeStruct((NW, RPW, 128), jnp.float32),
    in_specs=[
        pl.BlockSpec(memory_space=pltpu.SMEM),
        pl.BlockSpec(),
        pl.BlockSpec(),
        pl.BlockSpec(),
    ],
)


def _sc_body(tr_hbm, cw_hbm, ids_hbm, out0, out1,
             idx2, tv, cv, gt, sv, zv, nt, nc, tt_sh, tc_sh, sem):
    cid = lax.axis_index("c")
    sid = lax.axis_index("s")
    out_refs = [out0, out1]

    sa = pltpu.async_copy(tr_hbm.at[sid], tv, sem)
    sb = pltpu.async_copy(cw_hbm.at[sid], cv, sem)
    sa.wait()
    sb.wait()

    def _zb(i, c):
        zv[pl.ds(i * 16, 16)] = jnp.zeros((16,), jnp.float32)
        return c
    lax.fori_loop(0, ZPW // 16, _zb, 0)

    def _za(j, c):
        for kk in range(8):
            sv[j, pl.ds(kk * 16, 16)] = jnp.zeros((16,), jnp.float32)
        return c
    lax.fori_loop(0, RPW, _za, 0)

    def _round(r, carry):
        za = pltpu.async_copy(zv, tt_sh.at[pl.ds(sid * ZPW, ZPW)], sem)
        zb = pltpu.async_copy(zv, tc_sh.at[pl.ds(sid * ZPW, ZPW)], sem)
        za.wait()
        zb.wait()
        plsc.subcore_barrier()

        v = 3 * cid + r
        pltpu.sync_copy(ids_hbm.at[v, sid], idx2)

        def _sf(j, c):
            pltpu.async_copy(tv.at[j], tt_sh.at[idx2.at[j]], sem, add=True)
            pltpu.async_copy(cv.at[j], tc_sh.at[idx2.at[j]], sem, add=True)
            return c
        lax.fori_loop(0, RPW, _sf, 0)

        def _sd(j, c):
            pltpu.make_async_copy(tv.at[j], tt_sh.at[idx2.at[j]], sem).wait()
            pltpu.make_async_copy(cv.at[j], tc_sh.at[idx2.at[j]], sem).wait()
            return c
        lax.fori_loop(0, RPW, _sd, 0)
        plsc.subcore_barrier()

        na = pltpu.async_copy(tt_sh.at[pl.ds(sid * ZPW, ZPW)], nt, sem)
        nb = pltpu.async_copy(tc_sh.at[pl.ds(sid * ZPW, ZPW)], nc, sem)
        na.wait()
        nb.wait()

        def _nb(i, c):
            o = i * 16
            nt[pl.ds(o, 16)] = (nt[pl.ds(o, 16)]
                                / jnp.maximum(nc[pl.ds(o, 16)], 1.0))
            return c
        lax.fori_loop(0, ZPW // 16, _nb, 0)
        pltpu.sync_copy(nt, tt_sh.at[pl.ds(sid * ZPW, ZPW)])
        plsc.subcore_barrier()

        def _gf(j, c):
            pltpu.async_copy(tt_sh.at[idx2.at[j]], gt.at[j], sem)
            return c
        lax.fori_loop(0, RPW, _gf, 0)

        def _gd(j, c):
            pltpu.make_async_copy(tt_sh.at[idx2.at[j]], gt.at[j], sem).wait()
            return c
        lax.fori_loop(0, RPW, _gd, 0)

        def _ab(j, c):
            for kk in range(8):
                o = kk * 16
                sv[j, pl.ds(o, 16)] = (sv[j, pl.ds(o, 16)]
                                       + gt[j, pl.ds(o, 16)])
            return c
        lax.fori_loop(0, RPW, _ab, 0)
        plsc.subcore_barrier()
        return carry

    lax.fori_loop(0, 3, _round, 0)

    for c in range(2):
        @pl.when(cid == c)
        def _(c=c):
            pltpu.sync_copy(sv, out_refs[c].at[sid])


_sc_call = functools.partial(
    pl.kernel,
    out_type=[jax.ShapeDtypeStruct((NW, RPW, 128), jnp.float32),
              jax.ShapeDtypeStruct((NW, RPW, 128), jnp.float32)],
    mesh=plsc.VectorSubcoreMesh(core_axis_name="c", subcore_axis_name="s"),
    scratch_types=[
        pltpu.VMEM((RPW, 128), jnp.int32),
        pltpu.VMEM((RPW, 128), jnp.float32),
        pltpu.VMEM((RPW, 128), jnp.float32),
        pltpu.VMEM((RPW, 128), jnp.float32),
        pltpu.VMEM((RPW, 128), jnp.float32),
        pltpu.VMEM((ZPW,), jnp.float32),
        pltpu.VMEM((ZPW,), jnp.float32),
        pltpu.VMEM((ZPW,), jnp.float32),
        pltpu.VMEM_SHARED((G_PAD,), jnp.float32),
        pltpu.VMEM_SHARED((G_PAD,), jnp.float32),
        pltpu.SemaphoreType.DMA,
    ],
)(_sc_body)


def _finish_body(tr, s0, s1, su, ii, u0, u1, o):
    trans = tr[...]
    isf = ii[...]
    susc = su[...]
    logp = -(susc * (s0[...] + s1[...]))
    p = jnp.exp(logp)
    a0 = jnp.log(p + EPS)
    a1 = jnp.log(1.0 - p + EPS)
    g0 = -jnp.log(-jnp.log(u0[...] + EPS) + EPS)
    g1 = -jnp.log(-jnp.log(u1[...] + EPS) + EPS)
    arg = (a1 - a0 + g1 - g0) * 10.0
    soft1 = 1.0 / (1.0 + jnp.exp(-arg))
    new_inf = soft1 * (1.0 - isf)
    o[0] = trans
    o[1] = p
    o[2] = new_inf
    o[3] = jnp.maximum(0.0, susc - new_inf)
    new_isinf = isf + new_inf
    o[4] = new_isinf
    o[5] = new_isinf * (1.0 / (1.0 + jnp.exp(-(trans - 1.0))))


_finish_call = pl.pallas_call(
    _finish_body,
    out_shape=jax.ShapeDtypeStruct((6, NW, RPW, 128), jnp.float32),
)


def _pad3d(x):
    return jnp.pad(x, (0, NP - N)).reshape(NW, RPW, 128)


def kernel(susceptibility, infection_time, max_infectiousness, is_infected,
           company_ids, school_ids, household_ids, university_ids,
           leisure_ids, care_home_ids, now):
    now_f = jnp.asarray(now, jnp.float32).reshape(1, 1)
    isf = _pad3d(is_infected.astype(jnp.float32))
    it3 = _pad3d(infection_time)
    mi3 = _pad3d(max_infectiousness)
    su3 = _pad3d(susceptibility)
    ids4 = jnp.stack(
        [jnp.pad(i, (0, NP - N)) for i in
         (company_ids, school_ids, household_ids,
          university_ids, leisure_ids, care_home_ids)]
    ).astype(jnp.int32).reshape(6, NW, RPW, 128)
    cw3 = jnp.where(jnp.arange(NP) < N, 1.0, 0.0).astype(
        jnp.float32).reshape(NW, RPW, 128)
    u = jax.random.uniform(jax.random.key(42), (N, 2), dtype=jnp.float32)
    u0 = _pad3d(u[:, 0])
    u1 = _pad3d(u[:, 1])

    trans3 = _trans_call(now_f, it3, mi3, isf)
    s0, s1 = _sc_call(trans3, cw3, ids4)
    out = _finish_call(trans3, s0, s1, su3, isf, u0, u1)
    return out.reshape(6, NP)[:, :N]
